# baseline (speedup 1.0000x reference)
import functools

import jax
import jax.numpy as jnp
from jax import lax
from jax.experimental import pallas as pl
from jax.experimental.pallas import tpu as pltpu
from jax.experimental.pallas import tpu_sc as plsc


@functools.lru_cache(maxsize=None)
def _make_sc_gather(Vp, D, B):
  info = plsc.get_sparse_core_info()
  NC, NS, L = info.num_cores, info.num_subcores, info.num_lanes
  NW = NC * NS
  assert B % (8 * NW) == 0
  b_per_w = B // NW
  mesh = plsc.VectorSubcoreMesh(core_axis_name="c", subcore_axis_name="s")

  @functools.partial(
      pl.kernel,
      mesh=mesh,
      out_type=jax.ShapeDtypeStruct((B, 128), jnp.float32),
      scratch_types=[
          pltpu.VMEM((b_per_w,), jnp.int32),
          pltpu.VMEM((b_per_w,), jnp.int32),
          pltpu.VMEM((b_per_w, 128), jnp.float32),
          pltpu.SemaphoreType.DMA,
      ],
  )
  def gather(table_hbm, idx_hbm, out_hbm, idx_v, r_v, wide_v, sem):
    wid = lax.axis_index("s") * NC + lax.axis_index("c")
    base = wid * b_per_w
    pltpu.sync_copy(idx_hbm.at[pl.ds(base, b_per_w)], idx_v)
    for c in range(b_per_w // L):
      chunk = idx_v[pl.ds(c * L, L)]
      r_v[pl.ds(c * L, L)] = lax.shift_right_logical(chunk, 3)
    pltpu.async_copy(table_hbm.at[r_v], wide_v, sem).wait()
    pltpu.sync_copy(wide_v, out_hbm.at[pl.ds(base, b_per_w)])

  return gather


def _make_proj_body(D, nsel):
  def body(w_ref, wide_ref, sel_ref, o_ref, xe_s):
    @pl.when(pl.program_id(0) == 0)
    def _():
      sel = sel_ref[...]
      acc = jnp.zeros(xe_s.shape[:1] + (D,), jnp.float32)
      for s in range(nsel):
        acc += jnp.where(sel == s, wide_ref[:, s * D:(s + 1) * D],
                         jnp.zeros_like(acc))
      xe_s[:, :D] = acc
      xe_s[:, D:] = jnp.ones(xe_s.shape[:1] + (1,), jnp.float32)

    o_ref[...] = lax.dot_general(
        w_ref[...], xe_s[...],
        dimension_numbers=(((0,), (1,)), ((), ())),
        preferred_element_type=jnp.float32,
    )
  return body


@functools.lru_cache(maxsize=None)
def _make_projection(V, D, B, vblk):
  De = D + 1
  grid = (pl.cdiv(V, vblk),)
  return pl.pallas_call(
      _make_proj_body(D, 128 // D),
      grid=grid,
      in_specs=[
          pl.BlockSpec((De, vblk), lambda j: (0, j)),
          pl.BlockSpec((B, 128), lambda j: (0, 0)),
          pl.BlockSpec((B, 1), lambda j: (0, 0)),
      ],
      out_specs=pl.BlockSpec((vblk, B), lambda j: (j, 0)),
      out_shape=jax.ShapeDtypeStruct((V, B), jnp.float32),
      scratch_shapes=[pltpu.VMEM((B, De), jnp.float32)],
      compiler_params=pltpu.CompilerParams(
          dimension_semantics=("arbitrary",),
      ),
  )


def kernel(x, emb, W, b):
  V, D = emb.shape
  B = x.shape[0]
  xi = x.astype(jnp.int32)
  table = emb.reshape(V * D // 128, 128)
  wide = _make_sc_gather(V, D, B)(table, xi)
  sel = (xi & (128 // D - 1))[:, None]
  WeT = jnp.concatenate([W.T, b[None, :]], axis=0)
  outT = _make_projection(V, D, B, 3200)(WeT, wide, sel)
  return outT.T


# speedup vs baseline: 1.0697x; 1.0697x over previous
import functools

import jax
import jax.numpy as jnp
from jax import lax
from jax.experimental import pallas as pl
from jax.experimental.pallas import tpu as pltpu
from jax.experimental.pallas import tpu_sc as plsc


@functools.lru_cache(maxsize=None)
def _make_sc_gather(V, D, B):
  info = plsc.get_sparse_core_info()
  NC, NS, L = info.num_cores, info.num_subcores, info.num_lanes
  NW = NC * NS
  assert B % (8 * NW) == 0
  b_per_w = B // NW
  mesh = plsc.VectorSubcoreMesh(core_axis_name="c", subcore_axis_name="s")

  @functools.partial(
      pl.kernel,
      mesh=mesh,
      out_type=jax.ShapeDtypeStruct((D, B), jnp.float32),
      scratch_types=[
          pltpu.VMEM((b_per_w,), jnp.int32),
      ] + [pltpu.VMEM((b_per_w,), jnp.float32) for _ in range(D)] + [
          pltpu.SemaphoreType.DMA,
      ],
      compiler_params=pltpu.CompilerParams(use_tc_tiling_on_sc=False),
  )
  def gather(*args):
    cols_hbm = args[:D]
    idx_hbm = args[D]
    out_hbm = args[D + 1]
    idx_v = args[D + 2]
    col_v = args[D + 3:D + 3 + D]
    sem = args[D + 3 + D]
    wid = lax.axis_index("s") * NC + lax.axis_index("c")
    base = wid * b_per_w
    pltpu.sync_copy(idx_hbm.at[pl.ds(base, b_per_w)], idx_v)
    copies = [
        pltpu.async_copy(cols_hbm[k].at[idx_v], col_v[k], sem)
        for k in range(D)
    ]
    for c in copies:
      c.wait()
    for k in range(D):
      pltpu.sync_copy(col_v[k], out_hbm.at[k, pl.ds(base, b_per_w)])

  return gather


def _proj_body(w_ref, x_ref, o_ref):
  o_ref[...] = lax.dot_general(
      w_ref[...], x_ref[...],
      dimension_numbers=(((0,), (0,)), ((), ())),
      preferred_element_type=jnp.float32,
  )


@functools.lru_cache(maxsize=None)
def _make_projection(V, De, B, vblk):
  grid = (pl.cdiv(V, vblk),)
  return pl.pallas_call(
      _proj_body,
      grid=grid,
      in_specs=[
          pl.BlockSpec((De, vblk), lambda j: (0, j)),
          pl.BlockSpec((De, B), lambda j: (0, 0)),
      ],
      out_specs=pl.BlockSpec((vblk, B), lambda j: (j, 0)),
      out_shape=jax.ShapeDtypeStruct((V, B), jnp.float32),
      compiler_params=pltpu.CompilerParams(
          dimension_semantics=("arbitrary",),
      ),
  )


def kernel(x, emb, W, b):
  V, D = emb.shape
  B = x.shape[0]
  xi = x.astype(jnp.int32)
  cols = [emb[:, k] for k in range(D)]
  rowsT = _make_sc_gather(V, D, B)(*cols, xi)
  xeT = jnp.concatenate([rowsT, jnp.ones((1, B), jnp.float32)], axis=0)
  WeT = jnp.concatenate([W.T, b[None, :]], axis=0)
  outT = _make_projection(V, D + 1, B, 3200)(WeT, xeT)
  return outT.T


# speedup vs baseline: 1.2579x; 1.1760x over previous
import functools

import jax
import jax.numpy as jnp
from jax import lax
from jax.experimental import pallas as pl
from jax.experimental.pallas import tpu as pltpu
from jax.experimental.pallas import tpu_sc as plsc


@functools.lru_cache(maxsize=None)
def _make_sc_gather(V, D, B):
  info = plsc.get_sparse_core_info()
  NC, NS, L = info.num_cores, info.num_subcores, info.num_lanes
  NW = NC * NS
  assert B % (8 * NW) == 0
  b_per_w = B // NW
  mesh = plsc.VectorSubcoreMesh(core_axis_name="c", subcore_axis_name="s")

  @functools.partial(
      pl.kernel,
      mesh=mesh,
      out_type=jax.ShapeDtypeStruct((D, B), jnp.float32),
      scratch_types=[
          pltpu.VMEM((b_per_w,), jnp.int32),
      ] + [pltpu.VMEM((b_per_w,), jnp.float32) for _ in range(D)] + [
          pltpu.SemaphoreType.DMA,
      ],
      compiler_params=pltpu.CompilerParams(use_tc_tiling_on_sc=False),
  )
  def gather(*args):
    cols_hbm = args[:D]
    idx_hbm = args[D]
    out_hbm = args[D + 1]
    idx_v = args[D + 2]
    col_v = args[D + 3:D + 3 + D]
    sem = args[D + 3 + D]
    wid = lax.axis_index("s") * NC + lax.axis_index("c")
    base = wid * b_per_w
    pltpu.sync_copy(idx_hbm.at[pl.ds(base, b_per_w)], idx_v)
    copies = [
        pltpu.async_copy(cols_hbm[k].at[idx_v], col_v[k], sem)
        for k in range(D)
    ]
    for c in copies:
      c.wait()
    for k in range(D):
      pltpu.sync_copy(col_v[k], out_hbm.at[k, pl.ds(base, b_per_w)])

  return gather


@functools.lru_cache(maxsize=None)
def _make_col_split(V, D, vchunk):
  def body(t_ref, *out_refs):
    for k in range(D):
      out_refs[k][...] = t_ref[k, :]

  return pl.pallas_call(
      body,
      grid=(pl.cdiv(V, vchunk),),
      in_specs=[pl.BlockSpec((D, vchunk), lambda j: (0, j))],
      out_specs=[pl.BlockSpec((vchunk,), lambda j: (j,)) for _ in range(D)],
      out_shape=[jax.ShapeDtypeStruct((V,), jnp.float32) for _ in range(D)],
      compiler_params=pltpu.CompilerParams(
          dimension_semantics=("arbitrary",),
      ),
  )


def _proj_body(w_ref, x_ref, o_ref):
  o_ref[...] = lax.dot_general(
      w_ref[...], x_ref[...],
      dimension_numbers=(((0,), (0,)), ((), ())),
      preferred_element_type=jnp.float32,
  )


@functools.lru_cache(maxsize=None)
def _make_projection(V, De, B, vblk):
  grid = (pl.cdiv(V, vblk),)
  return pl.pallas_call(
      _proj_body,
      grid=grid,
      in_specs=[
          pl.BlockSpec((De, vblk), lambda j: (0, j)),
          pl.BlockSpec((De, B), lambda j: (0, 0)),
      ],
      out_specs=pl.BlockSpec((vblk, B), lambda j: (j, 0)),
      out_shape=jax.ShapeDtypeStruct((V, B), jnp.float32),
      compiler_params=pltpu.CompilerParams(
          dimension_semantics=("arbitrary",),
      ),
  )


def kernel(x, emb, W, b):
  V, D = emb.shape
  B = x.shape[0]
  xi = x.astype(jnp.int32)
  cols = _make_col_split(V, D, 25600)(emb.T)
  rowsT = _make_sc_gather(V, D, B)(*cols, xi)
  xeT = jnp.concatenate([rowsT, jnp.ones((1, B), jnp.float32)], axis=0)
  WeT = jnp.concatenate([W.T, b[None, :]], axis=0)
  outT = _make_projection(V, D + 1, B, 3200)(WeT, xeT)
  return outT.T
